# Initial kernel scaffold; baseline (speedup 1.0000x reference)
import jax
import jax.numpy as jnp
from jax.experimental import pallas as pl


def kernel(x, W1a, b1a, g1a, be1a, W1b, b1b, g1b, be1b, W2, b2, g2, be2, W3a, b3a, g3a, be3a, W3b, b3b, Wa1, ba1, ga1, bea1, Wa2, ba2):
    raise NotImplementedError("write your pallas kernel here")


# speedup vs baseline: 8.3373x; 8.3373x over previous
import functools

import jax
import jax.numpy as jnp
from jax import lax
from jax.experimental import pallas as pl
from jax.experimental.pallas import tpu as pltpu
from jax.experimental.pallas import tpu_sc as plsc

K = 16
Q = 256

_DN = (((1,), (1,)), ((), ()))


def _mm(a, b, precision=None):
    return lax.dot_general(a, b, _DN, precision=precision,
                           preferred_element_type=jnp.float32)


def _stage1_body(x_ref, w1a_ref, b1a_ref, g1a_ref, e1a_ref,
                 w1b_ref, b1b_ref, g1b_ref, e1b_ref,
                 w2_ref, b2_ref, g2_ref, e2_ref,
                 f1_ref, idx_ref, f2_ref):
    b = pl.program_id(0)
    qb = pl.program_id(1)
    x = x_ref[0]
    n = x.shape[1]
    xq = x_ref[0, :, pl.ds(qb * Q, Q)]
    xt = xq.T

    h = _mm(xt, w1a_ref[...]) + b1a_ref[...]
    h = jnp.maximum(h * g1a_ref[...] + e1a_ref[...], 0.0)
    f1 = _mm(h, w1b_ref[...]) + b1b_ref[...]
    f1 = jnp.maximum(f1 * g1b_ref[...] + e1b_ref[...], 0.0)
    f1_ref[0] = jnp.concatenate([f1, jnp.zeros((Q, 64), jnp.float32)], axis=1)

    d2 = jnp.sum(x * x, axis=0, keepdims=True)
    d2q = jnp.sum(xt * xt, axis=1, keepdims=True)
    ip = lax.dot_general(xq, x, (((0,), (0,)), ((), ())),
                         precision=lax.Precision.HIGHEST,
                         preferred_element_type=jnp.float32)
    dist = d2q - 2.0 * ip + d2

    iota = lax.broadcasted_iota(jnp.int32, (Q, n), 1)
    bign = jnp.int32(n)
    mask_acc = jnp.zeros((Q, n), jnp.float32)
    cols = []
    d = dist
    for k in range(K):
        m = jnp.min(d, axis=1, keepdims=True)
        j = jnp.min(jnp.where(d == m, iota, bign), axis=1, keepdims=True)
        onehot = iota == j
        mask_acc = mask_acc + onehot.astype(jnp.float32)
        d = jnp.where(onehot, jnp.inf, d)
        if k > 0:
            cols.append(j)
    idx15 = jnp.concatenate(cols, axis=1)
    idx_ref[0, 0] = idx15 + b * n

    p2 = jnp.concatenate(
        [x, x[0:1] * x, x[1:2] * x, x[2:3] * x], axis=0)
    sums = _mm(mask_acc, p2, precision=lax.Precision.HIGHEST)
    mean = sums[:, 0:3] * (1.0 / K)
    esq = sums[:, 3:12] * (1.0 / K)
    mm_ = jnp.concatenate([mean * mean[:, i:i + 1] for i in range(3)],
                          axis=1)
    cov9 = esq - mm_

    f2 = _mm(cov9, w2_ref[...]) + b2_ref[...]
    f2 = jnp.maximum(f2 * g2_ref[...] + e2_ref[...], 0.0)
    f2_ref[0] = f2


def _stage3_body(fk_ref, f1_ref, f2_ref, x_ref,
                 wa1_ref, ba1_ref, ga1_ref, ea1_ref, wa2_ref, ba2_ref,
                 w3a_ref, b3a_ref, g3a_ref, e3a_ref, w3b_ref, b3b_ref,
                 out_ref):
    f1 = f1_ref[0, :, :64]
    fk = fk_ref[0, 0][:, :64]
    fkd = fk.reshape(Q, K - 1, 64) - f1[:, None, :]

    a = _mm(fkd.reshape(Q * (K - 1), 64), wa1_ref[...]) + ba1_ref[...]
    a = jnp.maximum(a * ga1_ref[...] + ea1_ref[...], 0.0)
    a = _mm(a, wa2_ref[...]) + ba2_ref[...]
    a = a.reshape(Q, K - 1, 64)

    mx = a[:, 0, :]
    for k in range(1, K - 1):
        mx = jnp.maximum(mx, a[:, k, :])
    ssum = jnp.zeros((Q, 64), jnp.float32)
    agg = jnp.zeros((Q, 64), jnp.float32)
    for k in range(K - 1):
        e = jnp.exp(a[:, k, :] - mx)
        ssum = ssum + e
        agg = agg + e * fkd[:, k, :]
    f1_agg = agg / ssum

    f = jnp.concatenate([f1_agg, f2_ref[0]], axis=1)
    h3 = _mm(f, w3a_ref[...]) + b3a_ref[...]
    h3 = jnp.maximum(h3 * g3a_ref[...] + e3a_ref[...], 0.0)
    offt = lax.dot_general(w3b_ref[...], h3, _DN,
                           preferred_element_type=jnp.float32)
    offt = offt + b3b_ref[...]
    xq = x_ref[0]
    out_ref[0] = xq[:, None, :] + 0.15 * offt.reshape(3, 4, Q)


def _sc_gather(table, idx):
    nrow, d = table.shape
    nidx = idx.shape[0]
    info = plsc.get_sparse_core_info()
    nw = info.num_cores * info.num_subcores
    b_per_w = nidx // nw
    ch = 480
    nch = b_per_w // ch
    mesh = plsc.VectorSubcoreMesh(core_axis_name="c", subcore_axis_name="s")

    @functools.partial(
        pl.kernel, mesh=mesh,
        out_type=jax.ShapeDtypeStruct((nidx, d), jnp.float32),
        scratch_types=[
            pltpu.VMEM((ch,), jnp.int32),
            pltpu.VMEM((ch, d), jnp.float32),
            pltpu.SemaphoreType.DMA,
        ],
    )
    def k(table_hbm, idx_hbm, out_hbm, idx_v, rows_v, sem_g):
        wid = lax.axis_index("s") * info.num_cores + lax.axis_index("c")
        base = wid * b_per_w
        for i in range(nch):
            pltpu.sync_copy(idx_hbm.at[pl.ds(base + i * ch, ch)], idx_v)
            pltpu.async_copy(table_hbm.at[idx_v], rows_v, sem_g).wait()
            pltpu.sync_copy(rows_v, out_hbm.at[pl.ds(base + i * ch, ch)])

    return k(table, idx)


def kernel(x, W1a, b1a, g1a, be1a, W1b, b1b, g1b, be1b, W2, b2, g2, be2,
           W3a, b3a, g3a, be3a, W3b, b3b, Wa1, ba1, ga1, bea1, Wa2, ba2):
    B, C, N = x.shape
    nb = N // Q
    row = lambda v: v.reshape(1, -1)

    grid = (B, nb)
    wspec = lambda shp: pl.BlockSpec(shp, lambda b, q: (0,) * len(shp))

    f1, idxg, f2 = pl.pallas_call(
        _stage1_body,
        grid=grid,
        in_specs=[
            pl.BlockSpec((1, C, N), lambda b, q: (b, 0, 0)),
            wspec((32, 3)), wspec((1, 32)), wspec((1, 32)), wspec((1, 32)),
            wspec((64, 32)), wspec((1, 64)), wspec((1, 64)), wspec((1, 64)),
            wspec((32, 9)), wspec((1, 32)), wspec((1, 32)), wspec((1, 32)),
        ],
        out_specs=[
            pl.BlockSpec((1, Q, 128), lambda b, q: (b, q, 0)),
            pl.BlockSpec((1, 1, Q, K - 1), lambda b, q: (b, q, 0, 0)),
            pl.BlockSpec((1, Q, 32), lambda b, q: (b, q, 0)),
        ],
        out_shape=[
            jax.ShapeDtypeStruct((B, N, 128), jnp.float32),
            jax.ShapeDtypeStruct((B, nb, Q, K - 1), jnp.int32),
            jax.ShapeDtypeStruct((B, N, 32), jnp.float32),
        ],
    )(x, W1a, row(b1a), row(g1a), row(be1a),
      W1b, row(b1b), row(g1b), row(be1b),
      W2, row(b2), row(g2), row(be2))

    fk = _sc_gather(f1.reshape(B * N, 128), idxg.reshape(B * N * (K - 1)))

    out4 = pl.pallas_call(
        _stage3_body,
        grid=grid,
        in_specs=[
            pl.BlockSpec((1, 1, Q * (K - 1), 128), lambda b, q: (b, q, 0, 0)),
            pl.BlockSpec((1, Q, 128), lambda b, q: (b, q, 0)),
            pl.BlockSpec((1, Q, 32), lambda b, q: (b, q, 0)),
            pl.BlockSpec((1, C, Q), lambda b, q: (b, 0, q)),
            wspec((128, 64)), wspec((1, 128)), wspec((1, 128)), wspec((1, 128)),
            wspec((64, 128)), wspec((1, 64)),
            wspec((128, 96)), wspec((1, 128)), wspec((1, 128)), wspec((1, 128)),
            wspec((12, 128)), wspec((12, 1)),
        ],
        out_specs=pl.BlockSpec((1, C, 4, Q), lambda b, q: (b, 0, 0, q)),
        out_shape=jax.ShapeDtypeStruct((B, C, 4, N), jnp.float32),
    )(fk.reshape(B, nb, Q * (K - 1), 128), f1, f2, x,
      Wa1, row(ba1), row(ga1), row(bea1), Wa2, row(ba2),
      W3a, row(b3a), row(g3a), row(be3a), W3b, b3b.reshape(12, 1))

    return out4.reshape(B, C, 4 * N)


# speedup vs baseline: 10.1190x; 1.2137x over previous
import functools

import jax
import jax.numpy as jnp
from jax import lax
from jax.experimental import pallas as pl
from jax.experimental.pallas import tpu as pltpu
from jax.experimental.pallas import tpu_sc as plsc

K = 16
Q = 256

_DN = (((1,), (1,)), ((), ()))


def _mm(a, b, precision=None):
    return lax.dot_general(a, b, _DN, precision=precision,
                           preferred_element_type=jnp.float32)


def _stage1_body(x_ref, w1a_ref, c1a_ref, g1a_ref,
                 w1b_ref, c1b_ref, g1b_ref,
                 w2_ref, c2_ref, g2_ref,
                 f1_ref, idx_ref, f2_ref):
    b = pl.program_id(0)
    qb = pl.program_id(1)
    x = x_ref[0]
    n = x.shape[1]
    xq = x_ref[0, :, pl.ds(qb * Q, Q)]
    xt = xq.T

    h = jnp.maximum(_mm(xt, w1a_ref[...]) * g1a_ref[...] + c1a_ref[...], 0.0)
    f1 = jnp.maximum(_mm(h, w1b_ref[...]) * g1b_ref[...] + c1b_ref[...], 0.0)
    f1_ref[0] = jnp.concatenate([f1, jnp.zeros((Q, 64), jnp.float32)], axis=1)

    xx = x * x
    d2col = lax.dot_general(xx, jnp.ones((1, 3), jnp.float32),
                            (((0,), (1,)), ((), ())),
                            preferred_element_type=jnp.float32)
    d2row = jnp.sum(xq * xq, axis=0, keepdims=True)
    ipt = lax.dot_general(x, xq, (((0,), (0,)), ((), ())),
                          precision=lax.Precision.HIGHEST,
                          preferred_element_type=jnp.float32)
    dist = d2col - 2.0 * ipt + d2row

    iota = lax.broadcasted_iota(jnp.int32, (n, Q), 0)
    qrow = lax.broadcasted_iota(jnp.int32, (1, Q), 1) + qb * Q
    bign = jnp.int32(n)
    oh = iota == qrow
    mask_acc = oh.astype(jnp.float32)
    d = jnp.where(oh, jnp.inf, dist)
    rows = []
    for _ in range(K - 1):
        m = jnp.min(d, axis=0, keepdims=True)
        j = jnp.min(jnp.where(d == m, iota, bign), axis=0, keepdims=True)
        oh = iota == j
        mask_acc = mask_acc + oh.astype(jnp.float32)
        d = jnp.where(oh, jnp.inf, d)
        rows.append(j)
    idx15 = jnp.concatenate(rows, axis=0)
    idx_ref[0, 0] = idx15 + b * n

    p2 = jnp.concatenate(
        [x, x[0:1] * x, x[1:2] * x, x[2:3] * x], axis=0)
    sums = lax.dot_general(mask_acc, p2, (((0,), (1,)), ((), ())),
                           precision=lax.Precision.HIGHEST,
                           preferred_element_type=jnp.float32)
    mean = sums[:, 0:3] * (1.0 / K)
    esq = sums[:, 3:12] * (1.0 / K)
    mm_ = jnp.concatenate([mean * mean[:, i:i + 1] for i in range(3)],
                          axis=1)
    cov9 = esq - mm_

    f2 = jnp.maximum(_mm(cov9, w2_ref[...]) * g2_ref[...] + c2_ref[...], 0.0)
    f2_ref[0] = f2


def _stage3_body(fk_ref, f1_ref, f2_ref, x_ref,
                 wa1_ref, ca1_ref, ga1_ref, wa2_ref, ba2_ref,
                 w3a_ref, c3a_ref, g3a_ref, w3b_ref, b3b_ref,
                 out_ref):
    f1p = f1_ref[0]
    fk = fk_ref[0, 0]

    z = _mm(fk, wa1_ref[...])
    s1 = _mm(f1p, wa1_ref[...])
    a1 = []
    for k in range(K - 1):
        zk = z[k * Q:(k + 1) * Q, :] - s1
        a1.append(jnp.maximum(zk * ga1_ref[...] + ca1_ref[...], 0.0))
    a2 = _mm(jnp.concatenate(a1, axis=0), wa2_ref[...]) + ba2_ref[...]

    mx = a2[0:Q, :]
    for k in range(1, K - 1):
        mx = jnp.maximum(mx, a2[k * Q:(k + 1) * Q, :])
    ssum = jnp.zeros((Q, 128), jnp.float32)
    agg = jnp.zeros((Q, 128), jnp.float32)
    for k in range(K - 1):
        e = jnp.exp(a2[k * Q:(k + 1) * Q, :] - mx)
        ssum = ssum + e
        agg = agg + e * fk[k * Q:(k + 1) * Q, :]
    f1_agg = agg / ssum - f1p

    f = jnp.concatenate([f1_agg[:, :64], f2_ref[0]], axis=1)
    h3 = jnp.maximum(_mm(f, w3a_ref[...]) * g3a_ref[...] + c3a_ref[...], 0.0)
    offt = lax.dot_general(w3b_ref[...], h3, _DN,
                           preferred_element_type=jnp.float32)
    offt = offt + b3b_ref[...]
    xq = x_ref[0]
    out_ref[0] = xq[:, None, :] + 0.15 * offt.reshape(3, 4, Q)


def _sc_gather(table, idx):
    nrow, d = table.shape
    nidx = idx.shape[0]
    info = plsc.get_sparse_core_info()
    nw = info.num_cores * info.num_subcores
    b_per_w = nidx // nw
    ch = 480
    nch = b_per_w // ch
    mesh = plsc.VectorSubcoreMesh(core_axis_name="c", subcore_axis_name="s")

    @functools.partial(
        pl.kernel, mesh=mesh,
        out_type=jax.ShapeDtypeStruct((nidx, d), jnp.float32),
        scratch_types=[
            pltpu.VMEM((ch,), jnp.int32),
            pltpu.VMEM((ch,), jnp.int32),
            pltpu.VMEM((ch, d), jnp.float32),
            pltpu.VMEM((ch, d), jnp.float32),
            pltpu.SemaphoreType.DMA,
            pltpu.SemaphoreType.DMA,
            pltpu.SemaphoreType.DMA,
            pltpu.SemaphoreType.DMA,
            pltpu.SemaphoreType.DMA,
        ],
    )
    def k(table_hbm, idx_hbm, out_hbm, idx_a, idx_b, rows_a, rows_b,
          sem_ia, sem_ib, sem_g, sem_wa, sem_wb):
        wid = lax.axis_index("s") * info.num_cores + lax.axis_index("c")
        base = wid * b_per_w
        idx_v = [idx_a, idx_b]
        rows_v = [rows_a, rows_b]
        sem_i = [sem_ia, sem_ib]
        sem_w = [sem_wa, sem_wb]
        pltpu.async_copy(idx_hbm.at[pl.ds(base, ch)], idx_v[0], sem_i[0])
        for i in range(nch):
            s = i % 2
            o = (i + 1) % 2
            if i + 1 < nch:
                pltpu.async_copy(idx_hbm.at[pl.ds(base + (i + 1) * ch, ch)],
                                 idx_v[o], sem_i[o])
            pltpu.make_async_copy(idx_hbm.at[pl.ds(base + i * ch, ch)],
                                  idx_v[s], sem_i[s]).wait()
            if i >= 2:
                pltpu.make_async_copy(
                    rows_v[s], out_hbm.at[pl.ds(base + (i - 2) * ch, ch)],
                    sem_w[s]).wait()
            pltpu.async_copy(table_hbm.at[idx_v[s]], rows_v[s],
                             sem_g).wait()
            pltpu.async_copy(rows_v[s],
                             out_hbm.at[pl.ds(base + i * ch, ch)], sem_w[s])
        for i in (nch - 2, nch - 1):
            s = i % 2
            pltpu.make_async_copy(rows_v[s],
                                  out_hbm.at[pl.ds(base + i * ch, ch)],
                                  sem_w[s]).wait()

    return k(table, idx)


def kernel(x, W1a, b1a, g1a, be1a, W1b, b1b, g1b, be1b, W2, b2, g2, be2,
           W3a, b3a, g3a, be3a, W3b, b3b, Wa1, ba1, ga1, bea1, Wa2, ba2):
    B, C, N = x.shape
    nb = N // Q
    row = lambda v: v.reshape(1, -1)
    fold = lambda b_, g_, be_: (b_ * g_ + be_).reshape(1, -1)

    grid = (B, nb)
    wspec = lambda shp: pl.BlockSpec(shp, lambda b, q: (0,) * len(shp))

    f1, idxg, f2 = pl.pallas_call(
        _stage1_body,
        grid=grid,
        in_specs=[
            pl.BlockSpec((1, C, N), lambda b, q: (b, 0, 0)),
            wspec((32, 3)), wspec((1, 32)), wspec((1, 32)),
            wspec((64, 32)), wspec((1, 64)), wspec((1, 64)),
            wspec((32, 9)), wspec((1, 32)), wspec((1, 32)),
        ],
        out_specs=[
            pl.BlockSpec((1, Q, 128), lambda b, q: (b, q, 0)),
            pl.BlockSpec((1, 1, K - 1, Q), lambda b, q: (b, q, 0, 0)),
            pl.BlockSpec((1, Q, 32), lambda b, q: (b, q, 0)),
        ],
        out_shape=[
            jax.ShapeDtypeStruct((B, N, 128), jnp.float32),
            jax.ShapeDtypeStruct((B, nb, K - 1, Q), jnp.int32),
            jax.ShapeDtypeStruct((B, N, 32), jnp.float32),
        ],
    )(x, W1a, fold(b1a, g1a, be1a), row(g1a),
      W1b, fold(b1b, g1b, be1b), row(g1b),
      W2, fold(b2, g2, be2), row(g2))

    fk = _sc_gather(f1.reshape(B * N, 128), idxg.reshape(B * N * (K - 1)))

    wa1p = jnp.concatenate([Wa1, jnp.zeros((128, 64), jnp.float32)], axis=1)
    wa2p = jnp.concatenate([Wa2, jnp.zeros((64, 128), jnp.float32)], axis=0)
    ba2p = jnp.concatenate([ba2, jnp.zeros((64,), jnp.float32)])

    out4 = pl.pallas_call(
        _stage3_body,
        grid=grid,
        in_specs=[
            pl.BlockSpec((1, 1, (K - 1) * Q, 128), lambda b, q: (b, q, 0, 0)),
            pl.BlockSpec((1, Q, 128), lambda b, q: (b, q, 0)),
            pl.BlockSpec((1, Q, 32), lambda b, q: (b, q, 0)),
            pl.BlockSpec((1, C, Q), lambda b, q: (b, 0, q)),
            wspec((128, 128)), wspec((1, 128)), wspec((1, 128)),
            wspec((128, 128)), wspec((1, 128)),
            wspec((128, 96)), wspec((1, 128)), wspec((1, 128)),
            wspec((12, 128)), wspec((12, 1)),
        ],
        out_specs=pl.BlockSpec((1, C, 4, Q), lambda b, q: (b, 0, 0, q)),
        out_shape=jax.ShapeDtypeStruct((B, C, 4, N), jnp.float32),
    )(fk.reshape(B, nb, (K - 1) * Q, 128), f1, f2, x,
      wa1p, fold(ba1, ga1, bea1), row(ga1), wa2p, row(ba2p),
      W3a, fold(b3a, g3a, be3a), row(g3a), W3b, b3b.reshape(12, 1))

    return out4.reshape(B, C, 4 * N)


# speedup vs baseline: 13.0060x; 1.2853x over previous
import functools

import jax
import jax.numpy as jnp
from jax import lax
from jax.experimental import pallas as pl
from jax.experimental.pallas import tpu as pltpu
from jax.experimental.pallas import tpu_sc as plsc

K = 16
Q = 256

_DN = (((1,), (1,)), ((), ()))


def _mm(a, b, precision=None):
    return lax.dot_general(a, b, _DN, precision=precision,
                           preferred_element_type=jnp.float32)


def _stage1_body(x_ref, w1a_ref, c1a_ref, g1a_ref,
                 w1b_ref, c1b_ref, g1b_ref,
                 w2_ref, c2_ref, g2_ref,
                 f1_ref, idx_ref, f2_ref):
    b = pl.program_id(0)
    qb = pl.program_id(1)
    x = x_ref[0]
    n = x.shape[1]
    xq = x_ref[0, :, pl.ds(qb * Q, Q)]
    xt = xq.T

    h = jnp.maximum(_mm(xt, w1a_ref[...]) * g1a_ref[...] + c1a_ref[...], 0.0)
    f1 = jnp.maximum(_mm(h, w1b_ref[...]) * g1b_ref[...] + c1b_ref[...], 0.0)
    f1_ref[0] = jnp.concatenate([f1, jnp.zeros((Q, 64), jnp.float32)], axis=1)

    xx = x * x
    d2col = lax.dot_general(xx, jnp.ones((1, 3), jnp.float32),
                            (((0,), (1,)), ((), ())),
                            preferred_element_type=jnp.float32)
    d2row = jnp.sum(xq * xq, axis=0, keepdims=True)
    ipt = lax.dot_general(x, xq, (((0,), (0,)), ((), ())),
                          preferred_element_type=jnp.float32)
    dist = d2col - 2.0 * ipt + d2row

    iota = lax.broadcasted_iota(jnp.int32, (n, Q), 0)
    qrow = lax.broadcasted_iota(jnp.int32, (1, Q), 1) + qb * Q
    bign = jnp.int32(n)
    oh = iota == qrow
    mask_acc = oh.astype(jnp.int32)
    d = jnp.where(oh, jnp.inf, dist)
    rows = []
    for _ in range(K - 1):
        m = jnp.min(d, axis=0, keepdims=True)
        j = jnp.min(jnp.where(d == m, iota, bign), axis=0, keepdims=True)
        oh = iota == j
        mask_acc = mask_acc + oh.astype(jnp.int32)
        d = jnp.where(oh, jnp.inf, d)
        rows.append(j)
    idx15 = jnp.concatenate(rows, axis=0)
    idx_ref[0, 0] = idx15 + b * n

    p2 = jnp.concatenate(
        [x, x[0:1] * x, x[1:2] * x, x[2:3] * x], axis=0)
    sums = lax.dot_general(mask_acc.astype(jnp.float32), p2,
                           (((0,), (1,)), ((), ())),
                           preferred_element_type=jnp.float32)
    mean = sums[:, 0:3] * (1.0 / K)
    esq = sums[:, 3:12] * (1.0 / K)
    mm_ = jnp.concatenate([mean * mean[:, i:i + 1] for i in range(3)],
                          axis=1)
    cov9 = esq - mm_

    f2 = jnp.maximum(_mm(cov9, w2_ref[...]) * g2_ref[...] + c2_ref[...], 0.0)
    f2_ref[0] = f2


def _stage3_body(fk_ref, f1_ref, f2_ref, x_ref,
                 wa1_ref, ca1_ref, ga1_ref, wa2_ref, ba2_ref,
                 w3a_ref, c3a_ref, g3a_ref, w3b_ref, b3b_ref,
                 out_ref):
    f1p = f1_ref[0]
    fk = fk_ref[0, 0]

    z = _mm(fk, wa1_ref[...])
    s1 = _mm(f1p, wa1_ref[...])
    a1 = []
    for k in range(K - 1):
        zk = z[k * Q:(k + 1) * Q, :] - s1
        a1.append(jnp.maximum(zk * ga1_ref[...] + ca1_ref[...], 0.0))
    a2 = _mm(jnp.concatenate(a1, axis=0), wa2_ref[...]) + ba2_ref[...]

    mx = a2[0:Q, :]
    for k in range(1, K - 1):
        mx = jnp.maximum(mx, a2[k * Q:(k + 1) * Q, :])
    ssum = jnp.zeros((Q, 128), jnp.float32)
    agg = jnp.zeros((Q, 128), jnp.float32)
    for k in range(K - 1):
        e = jnp.exp(a2[k * Q:(k + 1) * Q, :] - mx)
        ssum = ssum + e
        agg = agg + e * fk[k * Q:(k + 1) * Q, :]
    f1_agg = agg / ssum - f1p

    f = jnp.concatenate([f1_agg[:, :64], f2_ref[0]], axis=1)
    h3 = jnp.maximum(_mm(f, w3a_ref[...]) * g3a_ref[...] + c3a_ref[...], 0.0)
    offt = lax.dot_general(w3b_ref[...], h3, _DN,
                           preferred_element_type=jnp.float32)
    offt = offt + b3b_ref[...]
    xq = x_ref[0]
    out_ref[0] = xq[:, None, :] + 0.15 * offt.reshape(3, 4, Q)


def _sc_gather(table, idx):
    nrow, d = table.shape
    nidx = idx.shape[0]
    info = plsc.get_sparse_core_info()
    nw = info.num_cores * info.num_subcores
    b_per_w = nidx // nw
    ch = 480
    nch = b_per_w // ch
    mesh = plsc.VectorSubcoreMesh(core_axis_name="c", subcore_axis_name="s")

    @functools.partial(
        pl.kernel, mesh=mesh,
        out_type=jax.ShapeDtypeStruct((nidx, d), jnp.float32),
        scratch_types=[
            pltpu.VMEM((ch,), jnp.int32),
            pltpu.VMEM((ch,), jnp.int32),
            pltpu.VMEM((ch, d), jnp.float32),
            pltpu.VMEM((ch, d), jnp.float32),
            pltpu.SemaphoreType.DMA,
            pltpu.SemaphoreType.DMA,
            pltpu.SemaphoreType.DMA,
            pltpu.SemaphoreType.DMA,
            pltpu.SemaphoreType.DMA,
        ],
    )
    def k(table_hbm, idx_hbm, out_hbm, idx_a, idx_b, rows_a, rows_b,
          sem_ia, sem_ib, sem_g, sem_wa, sem_wb):
        wid = lax.axis_index("s") * info.num_cores + lax.axis_index("c")
        base = wid * b_per_w
        idx_v = [idx_a, idx_b]
        rows_v = [rows_a, rows_b]
        sem_i = [sem_ia, sem_ib]
        sem_w = [sem_wa, sem_wb]
        pltpu.async_copy(idx_hbm.at[pl.ds(base, ch)], idx_v[0], sem_i[0])
        for i in range(nch):
            s = i % 2
            o = (i + 1) % 2
            if i + 1 < nch:
                pltpu.async_copy(idx_hbm.at[pl.ds(base + (i + 1) * ch, ch)],
                                 idx_v[o], sem_i[o])
            pltpu.make_async_copy(idx_hbm.at[pl.ds(base + i * ch, ch)],
                                  idx_v[s], sem_i[s]).wait()
            if i >= 2:
                pltpu.make_async_copy(
                    rows_v[s], out_hbm.at[pl.ds(base + (i - 2) * ch, ch)],
                    sem_w[s]).wait()
            pltpu.async_copy(table_hbm.at[idx_v[s]], rows_v[s],
                             sem_g).wait()
            pltpu.async_copy(rows_v[s],
                             out_hbm.at[pl.ds(base + i * ch, ch)], sem_w[s])
        for i in (nch - 2, nch - 1):
            s = i % 2
            pltpu.make_async_copy(rows_v[s],
                                  out_hbm.at[pl.ds(base + i * ch, ch)],
                                  sem_w[s]).wait()

    return k(table, idx)


def kernel(x, W1a, b1a, g1a, be1a, W1b, b1b, g1b, be1b, W2, b2, g2, be2,
           W3a, b3a, g3a, be3a, W3b, b3b, Wa1, ba1, ga1, bea1, Wa2, ba2):
    B, C, N = x.shape
    nb = N // Q
    row = lambda v: v.reshape(1, -1)
    fold = lambda b_, g_, be_: (b_ * g_ + be_).reshape(1, -1)

    grid = (B, nb)
    wspec = lambda shp: pl.BlockSpec(shp, lambda b, q: (0,) * len(shp))

    f1, idxg, f2 = pl.pallas_call(
        _stage1_body,
        grid=grid,
        in_specs=[
            pl.BlockSpec((1, C, N), lambda b, q: (b, 0, 0)),
            wspec((32, 3)), wspec((1, 32)), wspec((1, 32)),
            wspec((64, 32)), wspec((1, 64)), wspec((1, 64)),
            wspec((32, 9)), wspec((1, 32)), wspec((1, 32)),
        ],
        out_specs=[
            pl.BlockSpec((1, Q, 128), lambda b, q: (b, q, 0)),
            pl.BlockSpec((1, 1, K - 1, Q), lambda b, q: (b, q, 0, 0)),
            pl.BlockSpec((1, Q, 32), lambda b, q: (b, q, 0)),
        ],
        out_shape=[
            jax.ShapeDtypeStruct((B, N, 128), jnp.float32),
            jax.ShapeDtypeStruct((B, nb, K - 1, Q), jnp.int32),
            jax.ShapeDtypeStruct((B, N, 32), jnp.float32),
        ],
    )(x, W1a, fold(b1a, g1a, be1a), row(g1a),
      W1b, fold(b1b, g1b, be1b), row(g1b),
      W2, fold(b2, g2, be2), row(g2))

    fk = _sc_gather(f1.reshape(B * N, 128), idxg.reshape(B * N * (K - 1)))

    wa1p = jnp.concatenate([Wa1, jnp.zeros((128, 64), jnp.float32)], axis=1)
    wa2p = jnp.concatenate([Wa2, jnp.zeros((64, 128), jnp.float32)], axis=0)
    ba2p = jnp.concatenate([ba2, jnp.zeros((64,), jnp.float32)])

    out4 = pl.pallas_call(
        _stage3_body,
        grid=grid,
        in_specs=[
            pl.BlockSpec((1, 1, (K - 1) * Q, 128), lambda b, q: (b, q, 0, 0)),
            pl.BlockSpec((1, Q, 128), lambda b, q: (b, q, 0)),
            pl.BlockSpec((1, Q, 32), lambda b, q: (b, q, 0)),
            pl.BlockSpec((1, C, Q), lambda b, q: (b, 0, q)),
            wspec((128, 128)), wspec((1, 128)), wspec((1, 128)),
            wspec((128, 128)), wspec((1, 128)),
            wspec((128, 96)), wspec((1, 128)), wspec((1, 128)),
            wspec((12, 128)), wspec((12, 1)),
        ],
        out_specs=pl.BlockSpec((1, C, 4, Q), lambda b, q: (b, 0, 0, q)),
        out_shape=jax.ShapeDtypeStruct((B, C, 4, N), jnp.float32),
    )(fk.reshape(B, nb, (K - 1) * Q, 128), f1, f2, x,
      wa1p, fold(ba1, ga1, bea1), row(ga1), wa2p, row(ba2p),
      W3a, fold(b3a, g3a, be3a), row(g3a), W3b, b3b.reshape(12, 1))

    return out4.reshape(B, C, 4 * N)


# speedup vs baseline: 14.5903x; 1.1218x over previous
import functools

import jax
import jax.numpy as jnp
from jax import lax
from jax.experimental import pallas as pl
from jax.experimental.pallas import tpu as pltpu
from jax.experimental.pallas import tpu_sc as plsc

K = 16
Q = 256

_DN = (((1,), (1,)), ((), ()))


def _mm(a, b, precision=None):
    return lax.dot_general(a, b, _DN, precision=precision,
                           preferred_element_type=jnp.float32)


def _stage1_body(x_ref, w1a_ref, c1a_ref, g1a_ref,
                 w1b_ref, c1b_ref, g1b_ref,
                 w2_ref, c2_ref, g2_ref,
                 f1_ref, idx_ref, f2_ref):
    b = pl.program_id(0)
    qb = pl.program_id(1)
    x = x_ref[0]
    n = x.shape[1]
    xq = x_ref[0, :, pl.ds(qb * Q, Q)]
    xt = xq.T

    h = jnp.maximum(_mm(xt, w1a_ref[...]) * g1a_ref[...] + c1a_ref[...], 0.0)
    f1 = jnp.maximum(_mm(h, w1b_ref[...]) * g1b_ref[...] + c1b_ref[...], 0.0)
    f1_ref[0] = jnp.concatenate([f1, jnp.zeros((Q, 64), jnp.float32)], axis=1)

    d2q = jnp.sum(xt * xt, axis=1, keepdims=True)
    d2row = jnp.sum(x * x, axis=0, keepdims=True)
    ip = lax.dot_general(xq, x, (((0,), (0,)), ((), ())),
                         preferred_element_type=jnp.float32)
    dist = d2q - 2.0 * ip + d2row

    iota = lax.broadcasted_iota(jnp.int32, (Q, n), 1)
    qcol = lax.broadcasted_iota(jnp.int32, (Q, 1), 0) + qb * Q
    bign = jnp.int32(n)
    oh = iota == qcol
    mask_acc = oh.astype(jnp.int32)
    d = jnp.where(oh, jnp.inf, dist)
    cols = []
    for _ in range(K - 1):
        m = jnp.min(d, axis=1, keepdims=True)
        j = jnp.min(jnp.where(d == m, iota, bign), axis=1, keepdims=True)
        oh = iota == j
        mask_acc = mask_acc + oh.astype(jnp.int32)
        d = jnp.where(oh, jnp.inf, d)
        cols.append(j)
    idx15 = jnp.concatenate(cols, axis=1)
    idx_ref[0, 0] = idx15.T + b * n

    p2 = jnp.concatenate(
        [x, x[0:1] * x, x[1:2] * x, x[2:3] * x], axis=0)
    sums = lax.dot_general(mask_acc.astype(jnp.float32), p2,
                           (((1,), (1,)), ((), ())),
                           preferred_element_type=jnp.float32)
    mean = sums[:, 0:3] * (1.0 / K)
    esq = sums[:, 3:12] * (1.0 / K)
    mm_ = jnp.concatenate([mean * mean[:, i:i + 1] for i in range(3)],
                          axis=1)
    cov9 = esq - mm_

    f2 = jnp.maximum(_mm(cov9, w2_ref[...]) * g2_ref[...] + c2_ref[...], 0.0)
    f2_ref[0] = f2


def _stage3_body(fk_ref, f1_ref, f2_ref, x_ref,
                 wa1_ref, ca1_ref, ga1_ref, wa2_ref, ba2_ref,
                 w3a_ref, c3a_ref, g3a_ref, w3b_ref, b3b_ref,
                 out_ref):
    f1p = f1_ref[0]
    fk = fk_ref[0, 0]

    z = _mm(fk, wa1_ref[...])
    s1 = _mm(f1p, wa1_ref[...])
    a1 = []
    for k in range(K - 1):
        zk = z[k * Q:(k + 1) * Q, :] - s1
        a1.append(jnp.maximum(zk * ga1_ref[...] + ca1_ref[...], 0.0))
    a2 = _mm(jnp.concatenate(a1, axis=0), wa2_ref[...]) + ba2_ref[...]

    mx = a2[0:Q, :]
    for k in range(1, K - 1):
        mx = jnp.maximum(mx, a2[k * Q:(k + 1) * Q, :])
    ssum = jnp.zeros((Q, 128), jnp.float32)
    agg = jnp.zeros((Q, 128), jnp.float32)
    for k in range(K - 1):
        e = jnp.exp(a2[k * Q:(k + 1) * Q, :] - mx)
        ssum = ssum + e
        agg = agg + e * fk[k * Q:(k + 1) * Q, :]
    f1_agg = agg / ssum - f1p

    f = jnp.concatenate([f1_agg[:, :64], f2_ref[0]], axis=1)
    h3 = jnp.maximum(_mm(f, w3a_ref[...]) * g3a_ref[...] + c3a_ref[...], 0.0)
    offt = lax.dot_general(w3b_ref[...], h3, _DN,
                           preferred_element_type=jnp.float32)
    offt = offt + b3b_ref[...]
    xq = x_ref[0]
    out_ref[0] = xq[:, None, :] + 0.15 * offt.reshape(3, 4, Q)


def _sc_gather(table, idx):
    nrow, d = table.shape
    nidx = idx.shape[0]
    info = plsc.get_sparse_core_info()
    nw = info.num_cores * info.num_subcores
    b_per_w = nidx // nw
    ch = 480
    nch = b_per_w // ch
    mesh = plsc.VectorSubcoreMesh(core_axis_name="c", subcore_axis_name="s")

    @functools.partial(
        pl.kernel, mesh=mesh,
        out_type=jax.ShapeDtypeStruct((nidx, d), jnp.float32),
        scratch_types=[
            pltpu.VMEM((ch,), jnp.int32),
            pltpu.VMEM((ch,), jnp.int32),
            pltpu.VMEM((ch, d), jnp.float32),
            pltpu.VMEM((ch, d), jnp.float32),
            pltpu.SemaphoreType.DMA,
            pltpu.SemaphoreType.DMA,
            pltpu.SemaphoreType.DMA,
            pltpu.SemaphoreType.DMA,
            pltpu.SemaphoreType.DMA,
        ],
    )
    def k(table_hbm, idx_hbm, out_hbm, idx_a, idx_b, rows_a, rows_b,
          sem_ia, sem_ib, sem_g, sem_wa, sem_wb):
        wid = lax.axis_index("s") * info.num_cores + lax.axis_index("c")
        base = wid * b_per_w
        idx_v = [idx_a, idx_b]
        rows_v = [rows_a, rows_b]
        sem_i = [sem_ia, sem_ib]
        sem_w = [sem_wa, sem_wb]
        pltpu.async_copy(idx_hbm.at[pl.ds(base, ch)], idx_v[0], sem_i[0])
        for i in range(nch):
            s = i % 2
            o = (i + 1) % 2
            if i + 1 < nch:
                pltpu.async_copy(idx_hbm.at[pl.ds(base + (i + 1) * ch, ch)],
                                 idx_v[o], sem_i[o])
            pltpu.make_async_copy(idx_hbm.at[pl.ds(base + i * ch, ch)],
                                  idx_v[s], sem_i[s]).wait()
            if i >= 2:
                pltpu.make_async_copy(
                    rows_v[s], out_hbm.at[pl.ds(base + (i - 2) * ch, ch)],
                    sem_w[s]).wait()
            pltpu.async_copy(table_hbm.at[idx_v[s]], rows_v[s],
                             sem_g).wait()
            pltpu.async_copy(rows_v[s],
                             out_hbm.at[pl.ds(base + i * ch, ch)], sem_w[s])
        for i in (nch - 2, nch - 1):
            s = i % 2
            pltpu.make_async_copy(rows_v[s],
                                  out_hbm.at[pl.ds(base + i * ch, ch)],
                                  sem_w[s]).wait()

    return k(table, idx)


def kernel(x, W1a, b1a, g1a, be1a, W1b, b1b, g1b, be1b, W2, b2, g2, be2,
           W3a, b3a, g3a, be3a, W3b, b3b, Wa1, ba1, ga1, bea1, Wa2, ba2):
    B, C, N = x.shape
    nb = N // Q
    row = lambda v: v.reshape(1, -1)
    fold = lambda b_, g_, be_: (b_ * g_ + be_).reshape(1, -1)

    grid = (B, nb)
    wspec = lambda shp: pl.BlockSpec(shp, lambda b, q: (0,) * len(shp))

    f1, idxg, f2 = pl.pallas_call(
        _stage1_body,
        grid=grid,
        in_specs=[
            pl.BlockSpec((1, C, N), lambda b, q: (b, 0, 0)),
            wspec((32, 3)), wspec((1, 32)), wspec((1, 32)),
            wspec((64, 32)), wspec((1, 64)), wspec((1, 64)),
            wspec((32, 9)), wspec((1, 32)), wspec((1, 32)),
        ],
        out_specs=[
            pl.BlockSpec((1, Q, 128), lambda b, q: (b, q, 0)),
            pl.BlockSpec((1, 1, K - 1, Q), lambda b, q: (b, q, 0, 0)),
            pl.BlockSpec((1, Q, 32), lambda b, q: (b, q, 0)),
        ],
        out_shape=[
            jax.ShapeDtypeStruct((B, N, 128), jnp.float32),
            jax.ShapeDtypeStruct((B, nb, K - 1, Q), jnp.int32),
            jax.ShapeDtypeStruct((B, N, 32), jnp.float32),
        ],
    )(x, W1a, fold(b1a, g1a, be1a), row(g1a),
      W1b, fold(b1b, g1b, be1b), row(g1b),
      W2, fold(b2, g2, be2), row(g2))

    fk = _sc_gather(f1.reshape(B * N, 128), idxg.reshape(B * N * (K - 1)))

    wa1p = jnp.concatenate([Wa1, jnp.zeros((128, 64), jnp.float32)], axis=1)
    wa2p = jnp.concatenate([Wa2, jnp.zeros((64, 128), jnp.float32)], axis=0)
    ba2p = jnp.concatenate([ba2, jnp.zeros((64,), jnp.float32)])

    out4 = pl.pallas_call(
        _stage3_body,
        grid=grid,
        in_specs=[
            pl.BlockSpec((1, 1, (K - 1) * Q, 128), lambda b, q: (b, q, 0, 0)),
            pl.BlockSpec((1, Q, 128), lambda b, q: (b, q, 0)),
            pl.BlockSpec((1, Q, 32), lambda b, q: (b, q, 0)),
            pl.BlockSpec((1, C, Q), lambda b, q: (b, 0, q)),
            wspec((128, 128)), wspec((1, 128)), wspec((1, 128)),
            wspec((128, 128)), wspec((1, 128)),
            wspec((128, 96)), wspec((1, 128)), wspec((1, 128)),
            wspec((12, 128)), wspec((12, 1)),
        ],
        out_specs=pl.BlockSpec((1, C, 4, Q), lambda b, q: (b, 0, 0, q)),
        out_shape=jax.ShapeDtypeStruct((B, C, 4, N), jnp.float32),
    )(fk.reshape(B, nb, (K - 1) * Q, 128), f1, f2, x,
      wa1p, fold(ba1, ga1, bea1), row(ga1), wa2p, row(ba2p),
      W3a, fold(b3a, g3a, be3a), row(g3a), W3b, b3b.reshape(12, 1))

    return out4.reshape(B, C, 4 * N)


# speedup vs baseline: 17.6776x; 1.2116x over previous
import functools

import jax
import jax.numpy as jnp
from jax import lax
from jax.experimental import pallas as pl
from jax.experimental.pallas import tpu as pltpu
from jax.experimental.pallas import tpu_sc as plsc

K = 16
Q = 256

_DN = (((1,), (1,)), ((), ()))


def _mm(a, b, precision=None):
    return lax.dot_general(a, b, _DN, precision=precision,
                           preferred_element_type=jnp.float32)


def _stage1_body(x_ref, w1a_ref, c1a_ref, g1a_ref,
                 w1b_ref, c1b_ref, g1b_ref,
                 w2_ref, c2_ref, g2_ref,
                 f1_ref, idx_ref, f2_ref):
    b = pl.program_id(0)
    qb = pl.program_id(1)
    x = x_ref[0]
    n = x.shape[1]
    xq = x_ref[0, :, pl.ds(qb * Q, Q)]
    xt = xq.T

    h = jnp.maximum(_mm(xt, w1a_ref[...]) * g1a_ref[...] + c1a_ref[...], 0.0)
    f1 = jnp.maximum(_mm(h, w1b_ref[...]) * g1b_ref[...] + c1b_ref[...], 0.0)
    f1_ref[0] = jnp.concatenate([f1, jnp.zeros((Q, 64), jnp.float32)], axis=1)

    d2q = jnp.sum(xt * xt, axis=1, keepdims=True)
    d2row = jnp.sum(x * x, axis=0, keepdims=True)
    ip = lax.dot_general(xq, x, (((0,), (0,)), ((), ())),
                         preferred_element_type=jnp.float32)
    dist = d2q - 2.0 * ip + d2row

    iota = lax.broadcasted_iota(jnp.int32, (Q, n), 1)
    qcol = lax.broadcasted_iota(jnp.int32, (Q, 1), 0) + qb * Q
    maxi = jnp.int32(0x7FFFFFFF)
    bits = lax.bitcast_convert_type(jnp.maximum(dist, 0.0), jnp.int32)
    d = jnp.where(iota == qcol, maxi, (bits & ~jnp.int32(2047)) | iota)
    mask_acc = (iota == qcol).astype(jnp.int32)
    cols = []
    for _ in range(K - 1):
        m = jnp.min(d, axis=1, keepdims=True)
        oh = d == m
        mask_acc = mask_acc + oh.astype(jnp.int32)
        d = jnp.where(oh, maxi, d)
        cols.append(m & 2047)
    idx15 = jnp.concatenate(cols, axis=1)
    idx_ref[0, 0] = idx15.T + b * n

    p2 = jnp.concatenate(
        [x, x[0:1] * x, x[1:2] * x, x[2:3] * x], axis=0)
    sums = lax.dot_general(mask_acc.astype(jnp.float32), p2,
                           (((1,), (1,)), ((), ())),
                           preferred_element_type=jnp.float32)
    mean = sums[:, 0:3] * (1.0 / K)
    esq = sums[:, 3:12] * (1.0 / K)
    mm_ = jnp.concatenate([mean * mean[:, i:i + 1] for i in range(3)],
                          axis=1)
    cov9 = esq - mm_

    f2 = jnp.maximum(_mm(cov9, w2_ref[...]) * g2_ref[...] + c2_ref[...], 0.0)
    f2_ref[0] = f2


def _stage3_body(fk_ref, f1_ref, f2_ref, x_ref,
                 wa1_ref, ca1_ref, ga1_ref, wa2_ref, ba2_ref,
                 w3a_ref, c3a_ref, g3a_ref, w3b_ref, b3b_ref,
                 out_ref):
    f1p = f1_ref[0]
    fk = fk_ref[0, 0]

    z = _mm(fk, wa1_ref[...])
    s1 = _mm(f1p, wa1_ref[...])
    a1 = []
    for k in range(K - 1):
        zk = z[k * Q:(k + 1) * Q, :] - s1
        a1.append(jnp.maximum(zk * ga1_ref[...] + ca1_ref[...], 0.0))
    a2 = _mm(jnp.concatenate(a1, axis=0), wa2_ref[...]) + ba2_ref[...]

    mx = a2[0:Q, :]
    for k in range(1, K - 1):
        mx = jnp.maximum(mx, a2[k * Q:(k + 1) * Q, :])
    ssum = jnp.zeros((Q, 128), jnp.float32)
    agg = jnp.zeros((Q, 128), jnp.float32)
    for k in range(K - 1):
        e = jnp.exp(a2[k * Q:(k + 1) * Q, :] - mx)
        ssum = ssum + e
        agg = agg + e * fk[k * Q:(k + 1) * Q, :]
    f1_agg = agg / ssum - f1p

    f = jnp.concatenate([f1_agg[:, :64], f2_ref[0]], axis=1)
    h3 = jnp.maximum(_mm(f, w3a_ref[...]) * g3a_ref[...] + c3a_ref[...], 0.0)
    offt = lax.dot_general(w3b_ref[...], h3, _DN,
                           preferred_element_type=jnp.float32)
    offt = offt + b3b_ref[...]
    xq = x_ref[0]
    out_ref[0] = xq[:, None, :] + 0.15 * offt.reshape(3, 4, Q)


def _sc_gather(table, idx):
    nrow, d = table.shape
    nidx = idx.shape[0]
    info = plsc.get_sparse_core_info()
    nw = info.num_cores * info.num_subcores
    b_per_w = nidx // nw
    ch = 480
    nch = b_per_w // ch
    mesh = plsc.VectorSubcoreMesh(core_axis_name="c", subcore_axis_name="s")

    @functools.partial(
        pl.kernel, mesh=mesh,
        out_type=jax.ShapeDtypeStruct((nidx, d), jnp.float32),
        scratch_types=[
            pltpu.VMEM((ch,), jnp.int32),
            pltpu.VMEM((ch,), jnp.int32),
            pltpu.VMEM((ch, d), jnp.float32),
            pltpu.VMEM((ch, d), jnp.float32),
            pltpu.SemaphoreType.DMA,
            pltpu.SemaphoreType.DMA,
            pltpu.SemaphoreType.DMA,
            pltpu.SemaphoreType.DMA,
            pltpu.SemaphoreType.DMA,
        ],
    )
    def k(table_hbm, idx_hbm, out_hbm, idx_a, idx_b, rows_a, rows_b,
          sem_ia, sem_ib, sem_g, sem_wa, sem_wb):
        wid = lax.axis_index("s") * info.num_cores + lax.axis_index("c")
        base = wid * b_per_w
        idx_v = [idx_a, idx_b]
        rows_v = [rows_a, rows_b]
        sem_i = [sem_ia, sem_ib]
        sem_w = [sem_wa, sem_wb]
        pltpu.async_copy(idx_hbm.at[pl.ds(base, ch)], idx_v[0], sem_i[0])
        for i in range(nch):
            s = i % 2
            o = (i + 1) % 2
            if i + 1 < nch:
                pltpu.async_copy(idx_hbm.at[pl.ds(base + (i + 1) * ch, ch)],
                                 idx_v[o], sem_i[o])
            pltpu.make_async_copy(idx_hbm.at[pl.ds(base + i * ch, ch)],
                                  idx_v[s], sem_i[s]).wait()
            if i >= 2:
                pltpu.make_async_copy(
                    rows_v[s], out_hbm.at[pl.ds(base + (i - 2) * ch, ch)],
                    sem_w[s]).wait()
            pltpu.async_copy(table_hbm.at[idx_v[s]], rows_v[s],
                             sem_g).wait()
            pltpu.async_copy(rows_v[s],
                             out_hbm.at[pl.ds(base + i * ch, ch)], sem_w[s])
        for i in (nch - 2, nch - 1):
            s = i % 2
            pltpu.make_async_copy(rows_v[s],
                                  out_hbm.at[pl.ds(base + i * ch, ch)],
                                  sem_w[s]).wait()

    return k(table, idx)


def kernel(x, W1a, b1a, g1a, be1a, W1b, b1b, g1b, be1b, W2, b2, g2, be2,
           W3a, b3a, g3a, be3a, W3b, b3b, Wa1, ba1, ga1, bea1, Wa2, ba2):
    B, C, N = x.shape
    nb = N // Q
    row = lambda v: v.reshape(1, -1)
    fold = lambda b_, g_, be_: (b_ * g_ + be_).reshape(1, -1)

    grid = (B, nb)
    wspec = lambda shp: pl.BlockSpec(shp, lambda b, q: (0,) * len(shp))

    f1, idxg, f2 = pl.pallas_call(
        _stage1_body,
        grid=grid,
        in_specs=[
            pl.BlockSpec((1, C, N), lambda b, q: (b, 0, 0)),
            wspec((32, 3)), wspec((1, 32)), wspec((1, 32)),
            wspec((64, 32)), wspec((1, 64)), wspec((1, 64)),
            wspec((32, 9)), wspec((1, 32)), wspec((1, 32)),
        ],
        out_specs=[
            pl.BlockSpec((1, Q, 128), lambda b, q: (b, q, 0)),
            pl.BlockSpec((1, 1, K - 1, Q), lambda b, q: (b, q, 0, 0)),
            pl.BlockSpec((1, Q, 32), lambda b, q: (b, q, 0)),
        ],
        out_shape=[
            jax.ShapeDtypeStruct((B, N, 128), jnp.float32),
            jax.ShapeDtypeStruct((B, nb, K - 1, Q), jnp.int32),
            jax.ShapeDtypeStruct((B, N, 32), jnp.float32),
        ],
    )(x, W1a, fold(b1a, g1a, be1a), row(g1a),
      W1b, fold(b1b, g1b, be1b), row(g1b),
      W2, fold(b2, g2, be2), row(g2))

    fk = _sc_gather(f1.reshape(B * N, 128), idxg.reshape(B * N * (K - 1)))

    wa1p = jnp.concatenate([Wa1, jnp.zeros((128, 64), jnp.float32)], axis=1)
    wa2p = jnp.concatenate([Wa2, jnp.zeros((64, 128), jnp.float32)], axis=0)
    ba2p = jnp.concatenate([ba2, jnp.zeros((64,), jnp.float32)])

    out4 = pl.pallas_call(
        _stage3_body,
        grid=grid,
        in_specs=[
            pl.BlockSpec((1, 1, (K - 1) * Q, 128), lambda b, q: (b, q, 0, 0)),
            pl.BlockSpec((1, Q, 128), lambda b, q: (b, q, 0)),
            pl.BlockSpec((1, Q, 32), lambda b, q: (b, q, 0)),
            pl.BlockSpec((1, C, Q), lambda b, q: (b, 0, q)),
            wspec((128, 128)), wspec((1, 128)), wspec((1, 128)),
            wspec((128, 128)), wspec((1, 128)),
            wspec((128, 96)), wspec((1, 128)), wspec((1, 128)),
            wspec((12, 128)), wspec((12, 1)),
        ],
        out_specs=pl.BlockSpec((1, C, 4, Q), lambda b, q: (b, 0, 0, q)),
        out_shape=jax.ShapeDtypeStruct((B, C, 4, N), jnp.float32),
    )(fk.reshape(B, nb, (K - 1) * Q, 128), f1, f2, x,
      wa1p, fold(ba1, ga1, bea1), row(ga1), wa2p, row(ba2p),
      W3a, fold(b3a, g3a, be3a), row(g3a), W3b, b3b.reshape(12, 1))

    return out4.reshape(B, C, 4 * N)


# speedup vs baseline: 19.0343x; 1.0767x over previous
import functools

import jax
import jax.numpy as jnp
from jax import lax
from jax.experimental import pallas as pl
from jax.experimental.pallas import tpu as pltpu
from jax.experimental.pallas import tpu_sc as plsc

K = 16
Q = 256

_DN = (((1,), (1,)), ((), ()))


def _mm(a, b, precision=None):
    return lax.dot_general(a, b, _DN, precision=precision,
                           preferred_element_type=jnp.float32)


def _stage1_body(x_ref, w1a_ref, c1a_ref, g1a_ref,
                 w1b_ref, c1b_ref, g1b_ref,
                 w2_ref, c2_ref, g2_ref,
                 f1_ref, idx_ref, f2_ref):
    b = pl.program_id(0)
    qb = pl.program_id(1)
    x = x_ref[0]
    n = x.shape[1]
    xq = x_ref[0, :, pl.ds(qb * Q, Q)]
    xt = xq.T

    h = jnp.maximum(_mm(xt, w1a_ref[...]) * g1a_ref[...] + c1a_ref[...], 0.0)
    f1 = jnp.maximum(_mm(h, w1b_ref[...]) * g1b_ref[...] + c1b_ref[...], 0.0)
    f1_ref[0] = jnp.concatenate([f1, jnp.zeros((Q, 64), jnp.float32)], axis=1)

    d2q = jnp.sum(xt * xt, axis=1, keepdims=True)
    d2row = jnp.sum(x * x, axis=0, keepdims=True)
    ip = lax.dot_general(xq, x, (((0,), (0,)), ((), ())),
                         preferred_element_type=jnp.float32)
    dist = d2q - 2.0 * ip + d2row

    iota = lax.broadcasted_iota(jnp.int32, (Q, n), 1)
    qcol = lax.broadcasted_iota(jnp.int32, (Q, 1), 0) + qb * Q
    maxi = jnp.int32(0x7FFFFFFF)
    bits = lax.bitcast_convert_type(jnp.maximum(dist, 0.0), jnp.int32)
    d = jnp.where(iota == qcol, maxi, (bits & ~jnp.int32(2047)) | iota)
    mask_acc = (iota == qcol).astype(jnp.int32)
    cols = []
    for _ in range(K - 1):
        m = jnp.min(d, axis=1, keepdims=True)
        oh = d == m
        mask_acc = mask_acc + oh.astype(jnp.int32)
        d = jnp.where(oh, maxi, d)
        cols.append(m & 2047)
    idx15 = jnp.concatenate(cols, axis=1)
    idx_ref[0, 0] = idx15.T + b * n

    p2 = jnp.concatenate(
        [x, x[0:1] * x, x[1:2] * x, x[2:3] * x], axis=0)
    sums = lax.dot_general(mask_acc.astype(jnp.float32), p2,
                           (((1,), (1,)), ((), ())),
                           preferred_element_type=jnp.float32)
    mean = sums[:, 0:3] * (1.0 / K)
    esq = sums[:, 3:12] * (1.0 / K)
    mm_ = jnp.concatenate([mean * mean[:, i:i + 1] for i in range(3)],
                          axis=1)
    cov9 = esq - mm_

    f2 = jnp.maximum(_mm(cov9, w2_ref[...]) * g2_ref[...] + c2_ref[...], 0.0)
    f2_ref[0] = f2


def _stage3_body(fk_ref, f1_ref, f2_ref, x_ref,
                 wa1_ref, ca1_ref, ga1_ref, wa2_ref, ba2_ref,
                 w3a_ref, c3a_ref, g3a_ref, w3b_ref, b3b_ref,
                 out_ref):
    f1p = f1_ref[0]
    fk = fk_ref[0, 0]

    z = _mm(fk, wa1_ref[...])
    s1 = _mm(f1p, wa1_ref[...])
    a1 = []
    for k in range(K - 1):
        zk = z[k * Q:(k + 1) * Q, :] - s1
        a1.append(jnp.maximum(zk * ga1_ref[...] + ca1_ref[...], 0.0))
    a2 = _mm(jnp.concatenate(a1, axis=0), wa2_ref[...]) + ba2_ref[...]

    mx = a2[0:Q, :]
    for k in range(1, K - 1):
        mx = jnp.maximum(mx, a2[k * Q:(k + 1) * Q, :])
    ssum = jnp.zeros((Q, 128), jnp.float32)
    agg = jnp.zeros((Q, 128), jnp.float32)
    for k in range(K - 1):
        e = jnp.exp(a2[k * Q:(k + 1) * Q, :] - mx)
        ssum = ssum + e
        agg = agg + e * fk[k * Q:(k + 1) * Q, :]
    f1_agg = agg / ssum - f1p

    f = jnp.concatenate([f1_agg[:, :64], f2_ref[0]], axis=1)
    h3 = jnp.maximum(_mm(f, w3a_ref[...]) * g3a_ref[...] + c3a_ref[...], 0.0)
    offt = lax.dot_general(w3b_ref[...], h3, _DN,
                           preferred_element_type=jnp.float32)
    offt = offt + b3b_ref[...]
    xq = x_ref[0]
    out_ref[0] = xq[:, None, :] + 0.15 * offt.reshape(3, 4, Q)


def _sc_gather(table, idx):
    nrow, d = table.shape
    nidx = idx.shape[0]
    info = plsc.get_sparse_core_info()
    nw = info.num_cores * info.num_subcores
    b_per_w = nidx // nw
    ch = 480
    nch = b_per_w // ch
    mesh = plsc.VectorSubcoreMesh(core_axis_name="c", subcore_axis_name="s")

    @functools.partial(
        pl.kernel, mesh=mesh,
        out_type=jax.ShapeDtypeStruct((nidx, d), jnp.float32),
        scratch_types=[
            pltpu.VMEM((ch,), jnp.int32),
            pltpu.VMEM((ch,), jnp.int32),
            pltpu.VMEM((ch, d), jnp.float32),
            pltpu.VMEM((ch, d), jnp.float32),
            pltpu.SemaphoreType.DMA,
            pltpu.SemaphoreType.DMA,
            pltpu.SemaphoreType.DMA,
            pltpu.SemaphoreType.DMA,
            pltpu.SemaphoreType.DMA,
        ],
    )
    def k(table_hbm, idx_hbm, out_hbm, idx_a, idx_b, rows_a, rows_b,
          sem_ia, sem_ib, sem_g, sem_wa, sem_wb):
        wid = lax.axis_index("s") * info.num_cores + lax.axis_index("c")
        base = wid * b_per_w
        idx_v = [idx_a, idx_b]
        rows_v = [rows_a, rows_b]
        sem_i = [sem_ia, sem_ib]
        sem_w = [sem_wa, sem_wb]
        pltpu.async_copy(idx_hbm.at[pl.ds(base, ch)], idx_v[0], sem_i[0])
        for i in range(nch):
            s = i % 2
            o = (i + 1) % 2
            if i + 1 < nch:
                pltpu.async_copy(idx_hbm.at[pl.ds(base + (i + 1) * ch, ch)],
                                 idx_v[o], sem_i[o])
            pltpu.make_async_copy(idx_hbm.at[pl.ds(base + i * ch, ch)],
                                  idx_v[s], sem_i[s]).wait()
            if i >= 2:
                pltpu.make_async_copy(
                    rows_v[s], out_hbm.at[pl.ds(base + (i - 2) * ch, ch)],
                    sem_w[s]).wait()
            pltpu.async_copy(table_hbm.at[idx_v[s]], rows_v[s],
                             sem_g).wait()
            pltpu.async_copy(rows_v[s],
                             out_hbm.at[pl.ds(base + i * ch, ch)], sem_w[s])
        for i in (nch - 2, nch - 1):
            s = i % 2
            pltpu.make_async_copy(rows_v[s],
                                  out_hbm.at[pl.ds(base + i * ch, ch)],
                                  sem_w[s]).wait()

    return k(table, idx)


def kernel(x, W1a, b1a, g1a, be1a, W1b, b1b, g1b, be1b, W2, b2, g2, be2,
           W3a, b3a, g3a, be3a, W3b, b3b, Wa1, ba1, ga1, bea1, Wa2, ba2):
    B, C, N = x.shape
    nb = N // Q
    row = lambda v: v.reshape(1, -1)
    fold = lambda b_, g_, be_: (b_ * g_ + be_).reshape(1, -1)

    wspec = lambda shp: pl.BlockSpec(shp, lambda b, q: (0,) * len(shp))

    wa1p = jnp.concatenate([Wa1, jnp.zeros((128, 64), jnp.float32)], axis=1)
    wa2p = jnp.concatenate([Wa2, jnp.zeros((64, 128), jnp.float32)], axis=0)
    ba2p = jnp.concatenate([ba2, jnp.zeros((64,), jnp.float32)])

    def _stage1(xh):
        bh = xh.shape[0]
        return pl.pallas_call(
            _stage1_body,
            grid=(bh, nb),
            in_specs=[
                pl.BlockSpec((1, C, N), lambda b, q: (b, 0, 0)),
                wspec((32, 3)), wspec((1, 32)), wspec((1, 32)),
                wspec((64, 32)), wspec((1, 64)), wspec((1, 64)),
                wspec((32, 9)), wspec((1, 32)), wspec((1, 32)),
            ],
            out_specs=[
                pl.BlockSpec((1, Q, 128), lambda b, q: (b, q, 0)),
                pl.BlockSpec((1, 1, K - 1, Q), lambda b, q: (b, q, 0, 0)),
                pl.BlockSpec((1, Q, 32), lambda b, q: (b, q, 0)),
            ],
            out_shape=[
                jax.ShapeDtypeStruct((bh, N, 128), jnp.float32),
                jax.ShapeDtypeStruct((bh, nb, K - 1, Q), jnp.int32),
                jax.ShapeDtypeStruct((bh, N, 32), jnp.float32),
            ],
        )(xh, W1a, fold(b1a, g1a, be1a), row(g1a),
          W1b, fold(b1b, g1b, be1b), row(g1b),
          W2, fold(b2, g2, be2), row(g2))

    def _stage3(xh, fk, f1, f2):
        bh = xh.shape[0]
        return pl.pallas_call(
            _stage3_body,
            grid=(bh, nb),
            in_specs=[
                pl.BlockSpec((1, 1, (K - 1) * Q, 128),
                             lambda b, q: (b, q, 0, 0)),
                pl.BlockSpec((1, Q, 128), lambda b, q: (b, q, 0)),
                pl.BlockSpec((1, Q, 32), lambda b, q: (b, q, 0)),
                pl.BlockSpec((1, C, Q), lambda b, q: (b, 0, q)),
                wspec((128, 128)), wspec((1, 128)), wspec((1, 128)),
                wspec((128, 128)), wspec((1, 128)),
                wspec((128, 96)), wspec((1, 128)), wspec((1, 128)),
                wspec((12, 128)), wspec((12, 1)),
            ],
            out_specs=pl.BlockSpec((1, C, 4, Q), lambda b, q: (b, 0, 0, q)),
            out_shape=jax.ShapeDtypeStruct((bh, C, 4, N), jnp.float32),
        )(fk.reshape(bh, nb, (K - 1) * Q, 128), f1, f2, xh,
          wa1p, fold(ba1, ga1, bea1), row(ga1), wa2p, row(ba2p),
          W3a, fold(b3a, g3a, be3a), row(g3a), W3b, b3b.reshape(12, 1))

    halves = [x[:B // 2], x[B // 2:]]
    s1 = [_stage1(xh) for xh in halves]
    fks = [_sc_gather(f1.reshape(-1, 128), idxg.reshape(-1))
           for (f1, idxg, _) in s1]
    outs = [_stage3(xh, fk, f1, f2)
            for xh, fk, (f1, _, f2) in zip(halves, fks, s1)]
    return jnp.concatenate(outs, axis=0).reshape(B, C, 4 * N)


# speedup vs baseline: 19.2496x; 1.0113x over previous
import functools

import jax
import jax.numpy as jnp
from jax import lax
from jax.experimental import pallas as pl
from jax.experimental.pallas import tpu as pltpu
from jax.experimental.pallas import tpu_sc as plsc

K = 16
Q = 256

_DN = (((1,), (1,)), ((), ()))


def _mm(a, b, precision=None):
    return lax.dot_general(a, b, _DN, precision=precision,
                           preferred_element_type=jnp.float32)


def _stage1_body(x_ref, w1a_ref, c1a_ref, g1a_ref,
                 w1b_ref, c1b_ref, g1b_ref,
                 w2_ref, c2_ref, g2_ref,
                 f1_ref, idx_ref, f2_ref):
    b = pl.program_id(0)
    qb = pl.program_id(1)
    x = x_ref[0]
    n = x.shape[1]
    xq = x_ref[0, :, pl.ds(qb * Q, Q)]
    xt = xq.T

    h = jnp.maximum(_mm(xt, w1a_ref[...]) * g1a_ref[...] + c1a_ref[...], 0.0)
    f1 = jnp.maximum(_mm(h, w1b_ref[...]) * g1b_ref[...] + c1b_ref[...], 0.0)
    f1_ref[0] = jnp.concatenate([f1, jnp.zeros((Q, 64), jnp.float32)], axis=1)

    d2qrow = jnp.sum(xq * xq, axis=0, keepdims=True)
    d2row = jnp.sum(x * x, axis=0, keepdims=True)
    lhs5 = jnp.concatenate(
        [xq * -2.0, jnp.ones((1, Q), jnp.float32), d2qrow], axis=0)
    rhs5 = jnp.concatenate(
        [x, d2row, jnp.ones((1, n), jnp.float32)], axis=0)
    dist = lax.dot_general(lhs5, rhs5, (((0,), (0,)), ((), ())),
                           preferred_element_type=jnp.float32)

    iota = lax.broadcasted_iota(jnp.int32, (Q, n), 1)
    qcol = lax.broadcasted_iota(jnp.int32, (Q, 1), 0) + qb * Q
    maxi = jnp.int32(0x7FFFFFFF)
    bits = lax.bitcast_convert_type(dist, jnp.int32)
    d = jnp.where(iota == qcol, maxi, (bits & ~jnp.int32(2047)) | iota)
    mask_acc = (iota == qcol).astype(jnp.int32)
    cols = []
    for _ in range(K - 1):
        m = jnp.min(d, axis=1, keepdims=True)
        oh = d == m
        mask_acc = mask_acc + oh.astype(jnp.int32)
        d = jnp.where(oh, maxi, d)
        cols.append(m & 2047)
    idx15 = jnp.concatenate(cols, axis=1)
    idx_ref[0, 0] = idx15.T + b * n

    p2 = jnp.concatenate(
        [x, x[0:1] * x, x[1:2] * x, x[2:3] * x], axis=0)
    sums = lax.dot_general(mask_acc.astype(jnp.float32), p2,
                           (((1,), (1,)), ((), ())),
                           preferred_element_type=jnp.float32)
    mean = sums[:, 0:3] * (1.0 / K)
    esq = sums[:, 3:12] * (1.0 / K)
    mm_ = jnp.concatenate([mean * mean[:, i:i + 1] for i in range(3)],
                          axis=1)
    cov9 = esq - mm_

    f2 = jnp.maximum(_mm(cov9, w2_ref[...]) * g2_ref[...] + c2_ref[...], 0.0)
    f2_ref[0] = f2


def _stage3_body(fk_ref, f1_ref, f2_ref, x_ref,
                 wa1_ref, ca1_ref, ga1_ref, wa2_ref, ba2_ref,
                 w3a_ref, c3a_ref, g3a_ref, w3b_ref, b3b_ref,
                 out_ref):
    f1p = f1_ref[0]
    fk = fk_ref[0, 0]

    z = _mm(fk, wa1_ref[...])
    s1 = _mm(f1p, wa1_ref[...])
    a1 = []
    for k in range(K - 1):
        zk = z[k * Q:(k + 1) * Q, :] - s1
        a1.append(jnp.maximum(zk * ga1_ref[...] + ca1_ref[...], 0.0))
    a2 = _mm(jnp.concatenate(a1, axis=0), wa2_ref[...]) + ba2_ref[...]

    mx = a2[0:Q, :]
    for k in range(1, K - 1):
        mx = jnp.maximum(mx, a2[k * Q:(k + 1) * Q, :])
    ssum = jnp.zeros((Q, 128), jnp.float32)
    agg = jnp.zeros((Q, 128), jnp.float32)
    for k in range(K - 1):
        e = jnp.exp(a2[k * Q:(k + 1) * Q, :] - mx)
        ssum = ssum + e
        agg = agg + e * fk[k * Q:(k + 1) * Q, :]
    f1_agg = agg / ssum - f1p

    f = jnp.concatenate([f1_agg[:, :64], f2_ref[0]], axis=1)
    h3 = jnp.maximum(_mm(f, w3a_ref[...]) * g3a_ref[...] + c3a_ref[...], 0.0)
    offt = lax.dot_general(w3b_ref[...], h3, _DN,
                           preferred_element_type=jnp.float32)
    offt = offt + b3b_ref[...]
    xq = x_ref[0]
    out_ref[0] = xq[:, None, :] + 0.15 * offt.reshape(3, 4, Q)


def _sc_gather(table, idx):
    nrow, d = table.shape
    nidx = idx.shape[0]
    info = plsc.get_sparse_core_info()
    nw = info.num_cores * info.num_subcores
    b_per_w = nidx // nw
    ch = 480
    nch = b_per_w // ch
    mesh = plsc.VectorSubcoreMesh(core_axis_name="c", subcore_axis_name="s")

    @functools.partial(
        pl.kernel, mesh=mesh,
        out_type=jax.ShapeDtypeStruct((nidx, d), jnp.float32),
        scratch_types=[
            pltpu.VMEM((ch,), jnp.int32),
            pltpu.VMEM((ch,), jnp.int32),
            pltpu.VMEM((ch, d), jnp.float32),
            pltpu.VMEM((ch, d), jnp.float32),
            pltpu.SemaphoreType.DMA,
            pltpu.SemaphoreType.DMA,
            pltpu.SemaphoreType.DMA,
            pltpu.SemaphoreType.DMA,
            pltpu.SemaphoreType.DMA,
        ],
    )
    def k(table_hbm, idx_hbm, out_hbm, idx_a, idx_b, rows_a, rows_b,
          sem_ia, sem_ib, sem_g, sem_wa, sem_wb):
        wid = lax.axis_index("s") * info.num_cores + lax.axis_index("c")
        base = wid * b_per_w
        idx_v = [idx_a, idx_b]
        rows_v = [rows_a, rows_b]
        sem_i = [sem_ia, sem_ib]
        sem_w = [sem_wa, sem_wb]
        pltpu.async_copy(idx_hbm.at[pl.ds(base, ch)], idx_v[0], sem_i[0])
        for i in range(nch):
            s = i % 2
            o = (i + 1) % 2
            if i + 1 < nch:
                pltpu.async_copy(idx_hbm.at[pl.ds(base + (i + 1) * ch, ch)],
                                 idx_v[o], sem_i[o])
            pltpu.make_async_copy(idx_hbm.at[pl.ds(base + i * ch, ch)],
                                  idx_v[s], sem_i[s]).wait()
            if i >= 2:
                pltpu.make_async_copy(
                    rows_v[s], out_hbm.at[pl.ds(base + (i - 2) * ch, ch)],
                    sem_w[s]).wait()
            pltpu.async_copy(table_hbm.at[idx_v[s]], rows_v[s],
                             sem_g).wait()
            pltpu.async_copy(rows_v[s],
                             out_hbm.at[pl.ds(base + i * ch, ch)], sem_w[s])
        for i in (nch - 2, nch - 1):
            s = i % 2
            pltpu.make_async_copy(rows_v[s],
                                  out_hbm.at[pl.ds(base + i * ch, ch)],
                                  sem_w[s]).wait()

    return k(table, idx)


def kernel(x, W1a, b1a, g1a, be1a, W1b, b1b, g1b, be1b, W2, b2, g2, be2,
           W3a, b3a, g3a, be3a, W3b, b3b, Wa1, ba1, ga1, bea1, Wa2, ba2):
    B, C, N = x.shape
    nb = N // Q
    row = lambda v: v.reshape(1, -1)
    fold = lambda b_, g_, be_: (b_ * g_ + be_).reshape(1, -1)

    wspec = lambda shp: pl.BlockSpec(shp, lambda b, q: (0,) * len(shp))

    wa1p = jnp.concatenate([Wa1, jnp.zeros((128, 64), jnp.float32)], axis=1)
    wa2p = jnp.concatenate([Wa2, jnp.zeros((64, 128), jnp.float32)], axis=0)
    ba2p = jnp.concatenate([ba2, jnp.zeros((64,), jnp.float32)])

    def _stage1(xh):
        bh = xh.shape[0]
        return pl.pallas_call(
            _stage1_body,
            grid=(bh, nb),
            in_specs=[
                pl.BlockSpec((1, C, N), lambda b, q: (b, 0, 0)),
                wspec((32, 3)), wspec((1, 32)), wspec((1, 32)),
                wspec((64, 32)), wspec((1, 64)), wspec((1, 64)),
                wspec((32, 9)), wspec((1, 32)), wspec((1, 32)),
            ],
            out_specs=[
                pl.BlockSpec((1, Q, 128), lambda b, q: (b, q, 0)),
                pl.BlockSpec((1, 1, K - 1, Q), lambda b, q: (b, q, 0, 0)),
                pl.BlockSpec((1, Q, 32), lambda b, q: (b, q, 0)),
            ],
            out_shape=[
                jax.ShapeDtypeStruct((bh, N, 128), jnp.float32),
                jax.ShapeDtypeStruct((bh, nb, K - 1, Q), jnp.int32),
                jax.ShapeDtypeStruct((bh, N, 32), jnp.float32),
            ],
        )(xh, W1a, fold(b1a, g1a, be1a), row(g1a),
          W1b, fold(b1b, g1b, be1b), row(g1b),
          W2, fold(b2, g2, be2), row(g2))

    def _stage3(xh, fk, f1, f2):
        bh = xh.shape[0]
        return pl.pallas_call(
            _stage3_body,
            grid=(bh, nb),
            in_specs=[
                pl.BlockSpec((1, 1, (K - 1) * Q, 128),
                             lambda b, q: (b, q, 0, 0)),
                pl.BlockSpec((1, Q, 128), lambda b, q: (b, q, 0)),
                pl.BlockSpec((1, Q, 32), lambda b, q: (b, q, 0)),
                pl.BlockSpec((1, C, Q), lambda b, q: (b, 0, q)),
                wspec((128, 128)), wspec((1, 128)), wspec((1, 128)),
                wspec((128, 128)), wspec((1, 128)),
                wspec((128, 96)), wspec((1, 128)), wspec((1, 128)),
                wspec((12, 128)), wspec((12, 1)),
            ],
            out_specs=pl.BlockSpec((1, C, 4, Q), lambda b, q: (b, 0, 0, q)),
            out_shape=jax.ShapeDtypeStruct((bh, C, 4, N), jnp.float32),
        )(fk.reshape(bh, nb, (K - 1) * Q, 128), f1, f2, xh,
          wa1p, fold(ba1, ga1, bea1), row(ga1), wa2p, row(ba2p),
          W3a, fold(b3a, g3a, be3a), row(g3a), W3b, b3b.reshape(12, 1))

    halves = [x[:B // 2], x[B // 2:]]
    s1 = [_stage1(xh) for xh in halves]
    fks = [_sc_gather(f1.reshape(-1, 128), idxg.reshape(-1))
           for (f1, idxg, _) in s1]
    outs = [_stage3(xh, fk, f1, f2)
            for xh, fk, (f1, _, f2) in zip(halves, fks, s1)]
    return jnp.concatenate(outs, axis=0).reshape(B, C, 4 * N)
